# baseline (speedup 1.0000x reference)
import functools

import jax
import jax.numpy as jnp
from jax.experimental import pallas as pl
from jax.experimental.pallas import tpu as pltpu


def _conv_relu_kernel(x_ref, w_ref, o_ref, *, kh, kw, W, Wo, Ho, cin):
    Q = Ho * W
    xb = x_ref[0].astype(jnp.bfloat16)
    xb = jnp.concatenate(
        [xb, jnp.zeros((cin, 128), jnp.bfloat16)], axis=1)

    taps = []
    for ki in range(kh):
        for kj in range(kw):
            s = ki * W + kj
            taps.append(xb[:, s:s + Q])
    patch = jnp.concatenate(taps, axis=0)

    acc = jnp.dot(w_ref[...], patch, preferred_element_type=jnp.float32)
    acc = jnp.maximum(acc, 0.0)

    rows = [acc[:, h * W:h * W + Wo] for h in range(Ho)]
    o_ref[0] = jnp.concatenate(rows, axis=1)


@jax.jit
def _forward(x, weight):
    B, C_in, H, W = x.shape
    C_out, _, kh, kw = weight.shape
    Ho = H - kh + 1
    Wo = W - kw + 1
    HW = H * W

    w = jnp.transpose(weight.astype(jnp.bfloat16), (0, 2, 3, 1))
    w = w.reshape(C_out, kh * kw * C_in)

    x_flat = x.reshape(B, C_in, HW)

    body = functools.partial(_conv_relu_kernel, kh=kh, kw=kw, W=W, Wo=Wo,
                             Ho=Ho, cin=C_in)

    flops = 2 * B * C_out * (kh * kw * C_in) * Ho * W
    bytes_accessed = x_flat.size * 4 + w.size * 2 + B * C_out * Ho * Wo * 4

    out = pl.pallas_call(
        body,
        out_shape=jax.ShapeDtypeStruct((B, C_out, Ho * Wo), jnp.float32),
        grid_spec=pltpu.PrefetchScalarGridSpec(
            num_scalar_prefetch=0,
            grid=(B,),
            in_specs=[
                pl.BlockSpec((1, C_in, HW), lambda b: (b, 0, 0)),
                pl.BlockSpec((C_out, kh * kw * C_in), lambda b: (0, 0)),
            ],
            out_specs=pl.BlockSpec((1, C_out, Ho * Wo), lambda b: (b, 0, 0)),
        ),
        compiler_params=pltpu.CompilerParams(
            dimension_semantics=("parallel",),
            vmem_limit_bytes=64 * 1024 * 1024),
        cost_estimate=pl.CostEstimate(flops=flops, transcendentals=0,
                                      bytes_accessed=bytes_accessed),
    )(x_flat, w)

    return out.reshape(B, C_out, Ho, Wo)


def kernel(x, weight):
    return _forward(x, weight)


# speedup vs baseline: 1.4138x; 1.4138x over previous
import functools

import jax
import jax.numpy as jnp
from jax.experimental import pallas as pl
from jax.experimental.pallas import tpu as pltpu


def _conv_relu_kernel(x_ref, w_ref, o_ref, *, kh, kw, W, Q, cin):
    xb = jnp.transpose(x_ref[0], (1, 0)).astype(jnp.bfloat16)
    xb = jnp.concatenate(
        [xb, jnp.zeros((cin, 128), jnp.bfloat16)], axis=1)

    taps = []
    for ki in range(kh):
        for kj in range(kw):
            s = ki * W + kj
            taps.append(xb[:, s:s + Q])
    patch = jnp.concatenate(taps, axis=0)

    acc = jnp.dot(w_ref[...], patch, preferred_element_type=jnp.float32)
    o_ref[0] = jnp.maximum(acc, 0.0).astype(jnp.bfloat16)


@jax.jit
def _forward(x, weight):
    B, C_in, H, W = x.shape
    C_out, _, kh, kw = weight.shape
    Ho = H - kh + 1
    Wo = W - kw + 1
    Q = Ho * W
    HW = H * W

    w = jnp.transpose(weight.astype(jnp.bfloat16), (0, 2, 3, 1))
    w = w.reshape(C_out, kh * kw * C_in)

    xt = jnp.transpose(x, (0, 2, 3, 1)).reshape(B, HW, C_in)

    body = functools.partial(_conv_relu_kernel, kh=kh, kw=kw, W=W, Q=Q,
                             cin=C_in)

    flops = 2 * B * C_out * (kh * kw * C_in) * Q
    bytes_accessed = xt.size * 4 + w.size * 2 + B * C_out * Q * 2

    out = pl.pallas_call(
        body,
        out_shape=jax.ShapeDtypeStruct((B, C_out, Q), jnp.bfloat16),
        grid_spec=pltpu.PrefetchScalarGridSpec(
            num_scalar_prefetch=0,
            grid=(B,),
            in_specs=[
                pl.BlockSpec((1, HW, C_in), lambda b: (b, 0, 0)),
                pl.BlockSpec((C_out, kh * kw * C_in), lambda b: (0, 0)),
            ],
            out_specs=pl.BlockSpec((1, C_out, Q), lambda b: (b, 0, 0)),
        ),
        compiler_params=pltpu.CompilerParams(
            dimension_semantics=("parallel",),
            vmem_limit_bytes=64 * 1024 * 1024),
        cost_estimate=pl.CostEstimate(flops=flops, transcendentals=0,
                                      bytes_accessed=bytes_accessed),
    )(xt, w)

    y = out.reshape(B, C_out, Ho, W)[:, :, :, :Wo].astype(jnp.float32)
    return y


def kernel(x, weight):
    return _forward(x, weight)


# speedup vs baseline: 1.5327x; 1.0841x over previous
import functools

import jax
import jax.numpy as jnp
from jax.experimental import pallas as pl
from jax.experimental.pallas import tpu as pltpu


def _conv_relu_kernel(x_ref, w_ref, o_ref, *, kh, kw, W, Q, cin):
    xb = jnp.transpose(x_ref[0], (1, 0)).astype(jnp.bfloat16)
    xb = jnp.concatenate(
        [xb, jnp.zeros((cin, 128), jnp.bfloat16)], axis=1)

    taps = []
    for ki in range(kh):
        for kj in range(kw):
            s = ki * W + kj
            taps.append(xb[:, s:s + Q])
    patch = jnp.concatenate(taps, axis=0)

    acc = jnp.dot(w_ref[...], patch, preferred_element_type=jnp.float32)
    acc = jnp.maximum(acc, 0.0).astype(jnp.bfloat16)

    Wo = W - kw + 1
    Ho = Q // W
    rows = [acc[:, h * W:h * W + Wo] for h in range(Ho)]
    o_ref[0] = jnp.concatenate(rows, axis=1)


@jax.jit
def _forward(x, weight):
    B, C_in, H, W = x.shape
    C_out, _, kh, kw = weight.shape
    Ho = H - kh + 1
    Wo = W - kw + 1
    Q = Ho * W
    HW = H * W

    w = jnp.transpose(weight.astype(jnp.bfloat16), (0, 2, 3, 1))
    w = w.reshape(C_out, kh * kw * C_in)

    xt = jnp.transpose(x, (0, 2, 3, 1)).reshape(B, HW, C_in)

    body = functools.partial(_conv_relu_kernel, kh=kh, kw=kw, W=W, Q=Q,
                             cin=C_in)

    flops = 2 * B * C_out * (kh * kw * C_in) * Q
    bytes_accessed = xt.size * 4 + w.size * 2 + B * C_out * Q * 2

    out = pl.pallas_call(
        body,
        out_shape=jax.ShapeDtypeStruct((B, C_out, Ho * Wo), jnp.bfloat16),
        grid_spec=pltpu.PrefetchScalarGridSpec(
            num_scalar_prefetch=0,
            grid=(B,),
            in_specs=[
                pl.BlockSpec((1, HW, C_in), lambda b: (b, 0, 0)),
                pl.BlockSpec((C_out, kh * kw * C_in), lambda b: (0, 0)),
            ],
            out_specs=pl.BlockSpec((1, C_out, Ho * Wo), lambda b: (b, 0, 0)),
        ),
        compiler_params=pltpu.CompilerParams(
            dimension_semantics=("parallel",),
            vmem_limit_bytes=64 * 1024 * 1024),
        cost_estimate=pl.CostEstimate(flops=flops, transcendentals=0,
                                      bytes_accessed=bytes_accessed),
    )(xt, w)

    y = out.reshape(B, C_out, Ho, Wo).astype(jnp.float32)
    return y


def kernel(x, weight):
    return _forward(x, weight)


# speedup vs baseline: 1.5343x; 1.0011x over previous
import functools

import jax
import jax.numpy as jnp
from jax.experimental import pallas as pl
from jax.experimental.pallas import tpu as pltpu


def _conv_relu_kernel(x_ref, w_ref, o_ref, *, kh, kw, W, Q, cin, gb):
    Wo = W - kw + 1
    Ho = Q // W
    for g in range(gb):
        xb = jnp.transpose(x_ref[g], (1, 0)).astype(jnp.bfloat16)
        xb = jnp.concatenate(
            [xb, jnp.zeros((cin, 128), jnp.bfloat16)], axis=1)

        taps = []
        for ki in range(kh):
            for kj in range(kw):
                s = ki * W + kj
                taps.append(xb[:, s:s + Q])
        patch = jnp.concatenate(taps, axis=0)

        acc = jnp.dot(w_ref[...], patch, preferred_element_type=jnp.float32)
        acc = jnp.maximum(acc, 0.0).astype(jnp.bfloat16)

        rows = [acc[:, h * W:h * W + Wo] for h in range(Ho)]
        o_ref[g] = jnp.concatenate(rows, axis=1)


@jax.jit
def _forward(x, weight):
    B, C_in, H, W = x.shape
    C_out, _, kh, kw = weight.shape
    Ho = H - kh + 1
    Wo = W - kw + 1
    Q = Ho * W
    HW = H * W

    w = jnp.transpose(weight.astype(jnp.bfloat16), (0, 2, 3, 1))
    w = w.reshape(C_out, kh * kw * C_in)

    xt = jnp.transpose(x, (0, 2, 3, 1)).reshape(B, HW, C_in)

    GB = 1
    body = functools.partial(_conv_relu_kernel, kh=kh, kw=kw, W=W, Q=Q,
                             cin=C_in, gb=GB)

    flops = 2 * B * C_out * (kh * kw * C_in) * Q
    bytes_accessed = xt.size * 4 + w.size * 2 + B * C_out * Q * 2

    out = pl.pallas_call(
        body,
        out_shape=jax.ShapeDtypeStruct((B, C_out, Ho * Wo), jnp.bfloat16),
        grid_spec=pltpu.PrefetchScalarGridSpec(
            num_scalar_prefetch=0,
            grid=(B // GB,),
            in_specs=[
                pl.BlockSpec((GB, HW, C_in), lambda b: (b, 0, 0)),
                pl.BlockSpec((C_out, kh * kw * C_in), lambda b: (0, 0)),
            ],
            out_specs=pl.BlockSpec((GB, C_out, Ho * Wo), lambda b: (b, 0, 0)),
        ),
        compiler_params=pltpu.CompilerParams(
            dimension_semantics=("parallel",),
            vmem_limit_bytes=64 * 1024 * 1024),
        cost_estimate=pl.CostEstimate(flops=flops, transcendentals=0,
                                      bytes_accessed=bytes_accessed),
    )(xt, w)

    y = out.reshape(B, C_out, Ho, Wo).astype(jnp.float32)
    return y


def kernel(x, weight):
    return _forward(x, weight)
